# baseline (speedup 1.0000x reference)
import jax
import jax.numpy as jnp
from jax import lax
from jax.experimental import pallas as pl
from jax.experimental.pallas import tpu as pltpu
from jax.experimental.pallas import tpu_sc as plsc

B = 4096
F = 26
V = 100000
D = 16
L = 50

VP = 100352
VBLK = 50176
NVB = VP // VBLK
PROWS = F * VP // D

NC = 2
NS = 16
NW = NC * NS
EPW = B // NW
CH = 16
NCHUNK = EPW // CH
XROW = F + L

_SC_PARAMS = dict(
    compiler_params=pltpu.CompilerParams(
        needs_layout_passes=False, use_tc_tiling_on_sc=False))


def _p_body(t_ref, w_ref, o_ref):
    t = t_ref[0]
    w = w_ref[0]
    o_ref[...] = jnp.dot(w, t, preferred_element_type=jnp.float32
                         ).reshape(VBLK // 128, 128)


def _tc_p(tab_t, w3):
    return pl.pallas_call(
        _p_body,
        grid=(F, NVB),
        in_specs=[
            pl.BlockSpec((1, D, VBLK), lambda f, k: (f, 0, k)),
            pl.BlockSpec((1, 1, D), lambda f, k: (f, 0, 0)),
        ],
        out_specs=pl.BlockSpec((VBLK // 128, 128),
                               lambda f, k: (f * NVB + k, 0)),
        out_shape=jax.ShapeDtypeStruct((F * VP // 128, 128), jnp.float32),
    )(tab_t, w3)


def _q_body(t_ref, w_ref, o_ref):
    t = t_ref[...]
    w = w_ref[0]
    o_ref[...] = jnp.dot(w, t, preferred_element_type=jnp.float32
                         ).reshape(VBLK // 128, 128)


def _tc_q(var_t, w3):
    return pl.pallas_call(
        _q_body,
        grid=(NVB,),
        in_specs=[
            pl.BlockSpec((D, VBLK), lambda k: (0, k)),
            pl.BlockSpec((1, 1, D), lambda k: (F, 0, 0)),
        ],
        out_specs=pl.BlockSpec((VBLK // 128, 128), lambda k: (k, 0)),
        out_shape=jax.ShapeDtypeStruct((VP // 128, 128), jnp.float32),
    )(var_t, w3)


def _pool_body(x_hbm, q_hbm, sq_hbm, n0_hbm, xbuf, qbuf, sqbuf, n0buf, sem_q):
    wid = lax.axis_index("s") * NC + lax.axis_index("c")
    base = pl.multiple_of(wid * EPW, EPW)

    q_cp = pltpu.make_async_copy(q_hbm, qbuf, sem_q)
    q_cp.start()
    pltpu.sync_copy(x_hbm.at[:, pl.ds(base, EPW)], xbuf)
    q_cp.wait()

    lanes = lax.iota(jnp.int32, 16)

    def chunk(c, _):
        e0 = pl.multiple_of(c * CH, CH)
        elane = e0 + lanes

        def qstep(l, carry):
            sq, n0 = carry
            xv = plsc.load_gather(
                xbuf, [jnp.full((16,), F + l, jnp.int32), elane])
            val = plsc.load_gather(qbuf, [xv])
            live = xv != 0
            sq = sq + jnp.where(live, val, 0.0)
            n0 = n0 + jnp.where(live, 0.0, 1.0)
            return sq, n0
        sq, n0 = lax.fori_loop(
            0, L, qstep,
            (jnp.zeros((16,), jnp.float32), jnp.zeros((16,), jnp.float32)))
        sqbuf[pl.ds(e0, CH)] = sq
        n0buf[pl.ds(e0, CH)] = n0
        return _

    lax.fori_loop(0, NCHUNK, chunk, None)
    pltpu.sync_copy(sqbuf, sq_hbm.at[pl.ds(base, EPW)])
    pltpu.sync_copy(n0buf, n0_hbm.at[pl.ds(base, EPW)])


def _head_body(x_hbm, p_hbm, sq_hbm, n0_hbm, b_hbm, out_hbm,
               xbuf, bbuf, sqbuf, n0buf, sidx0, sidx1, srows0, srows1,
               outbuf, sem0, sem1):
    wid = lax.axis_index("s") * NC + lax.axis_index("c")
    base = pl.multiple_of(wid * EPW, EPW)

    pltpu.sync_copy(x_hbm.at[:, pl.ds(base, EPW)], xbuf)
    pltpu.sync_copy(sq_hbm.at[pl.ds(base, EPW)], sqbuf)
    pltpu.sync_copy(n0_hbm.at[pl.ds(base, EPW)], n0buf)
    pltpu.sync_copy(b_hbm, bbuf)

    lanes = lax.iota(jnp.int32, 16)
    bvec = bbuf[...]
    sidx = (sidx0, sidx1)
    srows = (srows0, srows1)
    sems = (sem0, sem1)

    def build(c):
        elane = c * CH + lanes

        def sfill(f, _):
            xv = plsc.load_gather(xbuf, [jnp.full((16,), f, jnp.int32), elane])
            sidx[c % 2][pl.ds(pl.multiple_of(f * CH, CH), CH)] = \
                f * (VP // D) + lax.shift_right_logical(xv, 4)
            return _
        lax.fori_loop(0, F, sfill, None)
        cp = pltpu.make_async_copy(p_hbm.at[sidx[c % 2]], srows[c % 2],
                                   sems[c % 2])
        cp.start()
        return cp

    cps = [None, None]
    cps[0] = build(0)
    for c in range(NCHUNK):
        if c + 1 < NCHUNK:
            cps[(c + 1) % 2] = build(c + 1)
        e0 = pl.multiple_of(c * CH, CH)
        elane = e0 + lanes

        cps[c % 2].wait()

        def pstep(f, sp):
            xv = plsc.load_gather(xbuf, [jnp.full((16,), f, jnp.int32), elane])
            val = plsc.load_gather(
                srows[c % 2], [f * CH + lanes, jnp.bitwise_and(xv, D - 1)])
            return sp + val
        sp = lax.fori_loop(0, F, pstep, jnp.zeros((16,), jnp.float32))

        cnt = jnp.float32(L) - n0buf[pl.ds(e0, CH)]
        logit = sp + sqbuf[pl.ds(e0, CH)] / (cnt + 1e-8) + bvec
        outbuf[pl.ds(e0, CH)] = 1.0 / (1.0 + jnp.exp(-logit))

    pltpu.sync_copy(outbuf, out_hbm.at[pl.ds(base, EPW)])


@jax.jit
def _run(x_t, tab_t, var_t, w2, b16):
    w3 = w2.reshape(F + 1, 1, D)
    q = _tc_q(var_t, w3).reshape(VP)

    mesh = plsc.VectorSubcoreMesh(core_axis_name="c", subcore_axis_name="s")
    pool = pl.kernel(
        _pool_body,
        out_type=(jax.ShapeDtypeStruct((B,), jnp.float32),
                  jax.ShapeDtypeStruct((B,), jnp.float32)),
        mesh=mesh,
        scratch_types=[
            pltpu.VMEM((XROW, EPW), jnp.int32),
            pltpu.VMEM((VP,), jnp.float32),
            pltpu.VMEM((EPW,), jnp.float32),
            pltpu.VMEM((EPW,), jnp.float32),
            pltpu.SemaphoreType.DMA,
        ],
        **_SC_PARAMS,
    )
    sq, n0 = pool(x_t, q)

    p = _tc_p(tab_t, w3).reshape(PROWS, D)

    head = pl.kernel(
        _head_body,
        out_type=jax.ShapeDtypeStruct((B,), jnp.float32),
        mesh=mesh,
        scratch_types=[
            pltpu.VMEM((XROW, EPW), jnp.int32),
            pltpu.VMEM((16,), jnp.float32),
            pltpu.VMEM((EPW,), jnp.float32),
            pltpu.VMEM((EPW,), jnp.float32),
            pltpu.VMEM((F * CH,), jnp.int32),
            pltpu.VMEM((F * CH,), jnp.int32),
            pltpu.VMEM((F * CH, D), jnp.float32),
            pltpu.VMEM((F * CH, D), jnp.float32),
            pltpu.VMEM((EPW,), jnp.float32),
            pltpu.SemaphoreType.DMA,
            pltpu.SemaphoreType.DMA,
        ],
        **_SC_PARAMS,
    )
    return head(x_t, p, sq, n0, b16)


def kernel(X, tables, var_table, W, b):
    tab_t = jnp.transpose(tables, (0, 2, 1))
    var_t = var_table.T
    x_t = X.T
    w2 = W.reshape(F + 1, D)
    b16 = jnp.broadcast_to(b.astype(jnp.float32), (16,))
    out = _run(x_t, tab_t, var_t, w2, b16)
    return out.reshape(B, 1)


# speedup vs baseline: 1.1095x; 1.1095x over previous
import jax
import jax.numpy as jnp
from jax import lax
from jax.experimental import pallas as pl
from jax.experimental.pallas import tpu as pltpu
from jax.experimental.pallas import tpu_sc as plsc

B = 4096
F = 26
V = 100000
D = 16
L = 50

VP = 100352
VBLK = 100352
NVB = VP // VBLK
PROWS = F * VP // D

NC = 2
NS = 16
NW = NC * NS
EPW = B // NW
CH = 16
NCHUNK = EPW // CH
XROW = F + L

_SC_PARAMS = dict(
    compiler_params=pltpu.CompilerParams(
        needs_layout_passes=False, use_tc_tiling_on_sc=False))


def _p_body(t_ref, w_ref, o_ref):
    t = t_ref[0]
    w = w_ref[0]
    o_ref[...] = jnp.dot(w, t, preferred_element_type=jnp.float32
                         ).reshape(VBLK // 128, 128)


def _tc_p(tab_t, w3):
    return pl.pallas_call(
        _p_body,
        grid=(F, NVB),
        in_specs=[
            pl.BlockSpec((1, D, VBLK), lambda f, k: (f, 0, k)),
            pl.BlockSpec((1, 1, D), lambda f, k: (f, 0, 0)),
        ],
        out_specs=pl.BlockSpec((VBLK // 128, 128),
                               lambda f, k: (f * NVB + k, 0)),
        out_shape=jax.ShapeDtypeStruct((F * VP // 128, 128), jnp.float32),
    )(tab_t, w3)


def _q_body(t_ref, w_ref, o_ref):
    t = t_ref[...]
    w = w_ref[0]
    o_ref[...] = jnp.dot(w, t, preferred_element_type=jnp.float32
                         ).reshape(VBLK // 128, 128)


def _tc_q(var_t, w3):
    return pl.pallas_call(
        _q_body,
        grid=(NVB,),
        in_specs=[
            pl.BlockSpec((D, VBLK), lambda k: (0, k)),
            pl.BlockSpec((1, 1, D), lambda k: (F, 0, 0)),
        ],
        out_specs=pl.BlockSpec((VBLK // 128, 128), lambda k: (k, 0)),
        out_shape=jax.ShapeDtypeStruct((VP // 128, 128), jnp.float32),
    )(var_t, w3)


def _pool_body(x_hbm, q_hbm, sq_hbm, n0_hbm, xbuf, qbuf, sqbuf, n0buf, sem_q):
    wid = lax.axis_index("s") * NC + lax.axis_index("c")
    base = pl.multiple_of(wid * EPW, EPW)

    q_cp = pltpu.make_async_copy(q_hbm, qbuf, sem_q)
    q_cp.start()
    pltpu.sync_copy(x_hbm.at[:, pl.ds(base, EPW)], xbuf)
    q_cp.wait()

    lanes = lax.iota(jnp.int32, 16)

    def chunk(c, _):
        e0 = pl.multiple_of(c * CH, CH)
        elane = e0 + lanes

        def qstep(l, carry):
            sq, n0 = carry
            xv = plsc.load_gather(
                xbuf, [jnp.full((16,), F + l, jnp.int32), elane])
            val = plsc.load_gather(qbuf, [xv])
            live = xv != 0
            sq = sq + jnp.where(live, val, 0.0)
            n0 = n0 + jnp.where(live, 0.0, 1.0)
            return sq, n0
        sq, n0 = lax.fori_loop(
            0, L, qstep,
            (jnp.zeros((16,), jnp.float32), jnp.zeros((16,), jnp.float32)))
        sqbuf[pl.ds(e0, CH)] = sq
        n0buf[pl.ds(e0, CH)] = n0
        return _

    lax.fori_loop(0, NCHUNK, chunk, None)
    pltpu.sync_copy(sqbuf, sq_hbm.at[pl.ds(base, EPW)])
    pltpu.sync_copy(n0buf, n0_hbm.at[pl.ds(base, EPW)])


def _head_body(x_hbm, p_hbm, sq_hbm, n0_hbm, b_hbm, out_hbm,
               xbuf, bbuf, sqbuf, n0buf, sidx0, sidx1, srows0, srows1,
               outbuf, sem0, sem1):
    wid = lax.axis_index("s") * NC + lax.axis_index("c")
    base = pl.multiple_of(wid * EPW, EPW)

    pltpu.sync_copy(x_hbm.at[:, pl.ds(base, EPW)], xbuf)
    pltpu.sync_copy(sq_hbm.at[pl.ds(base, EPW)], sqbuf)
    pltpu.sync_copy(n0_hbm.at[pl.ds(base, EPW)], n0buf)
    pltpu.sync_copy(b_hbm, bbuf)

    lanes = lax.iota(jnp.int32, 16)
    bvec = bbuf[...]
    sidx = (sidx0, sidx1)
    srows = (srows0, srows1)
    sems = (sem0, sem1)

    def build(c):
        elane = c * CH + lanes

        def sfill(f, _):
            xv = plsc.load_gather(xbuf, [jnp.full((16,), f, jnp.int32), elane])
            sidx[c % 2][pl.ds(pl.multiple_of(f * CH, CH), CH)] = \
                f * (VP // D) + lax.shift_right_logical(xv, 4)
            return _
        lax.fori_loop(0, F, sfill, None)
        cp = pltpu.make_async_copy(p_hbm.at[sidx[c % 2]], srows[c % 2],
                                   sems[c % 2])
        cp.start()
        return cp

    cps = [None, None]
    cps[0] = build(0)
    for c in range(NCHUNK):
        if c + 1 < NCHUNK:
            cps[(c + 1) % 2] = build(c + 1)
        e0 = pl.multiple_of(c * CH, CH)
        elane = e0 + lanes

        cps[c % 2].wait()

        def pstep(f, sp):
            xv = plsc.load_gather(xbuf, [jnp.full((16,), f, jnp.int32), elane])
            val = plsc.load_gather(
                srows[c % 2], [f * CH + lanes, jnp.bitwise_and(xv, D - 1)])
            return sp + val
        sp = lax.fori_loop(0, F, pstep, jnp.zeros((16,), jnp.float32))

        cnt = jnp.float32(L) - n0buf[pl.ds(e0, CH)]
        logit = sp + sqbuf[pl.ds(e0, CH)] / (cnt + 1e-8) + bvec
        outbuf[pl.ds(e0, CH)] = 1.0 / (1.0 + jnp.exp(-logit))

    pltpu.sync_copy(outbuf, out_hbm.at[pl.ds(base, EPW)])


@jax.jit
def _run(x_t, tab_t, var_t, w2, b16):
    w3 = w2.reshape(F + 1, 1, D)
    q = _tc_q(var_t, w3).reshape(VP)

    mesh = plsc.VectorSubcoreMesh(core_axis_name="c", subcore_axis_name="s")
    pool = pl.kernel(
        _pool_body,
        out_type=(jax.ShapeDtypeStruct((B,), jnp.float32),
                  jax.ShapeDtypeStruct((B,), jnp.float32)),
        mesh=mesh,
        scratch_types=[
            pltpu.VMEM((XROW, EPW), jnp.int32),
            pltpu.VMEM((VP,), jnp.float32),
            pltpu.VMEM((EPW,), jnp.float32),
            pltpu.VMEM((EPW,), jnp.float32),
            pltpu.SemaphoreType.DMA,
        ],
        **_SC_PARAMS,
    )
    sq, n0 = pool(x_t, q)

    p = _tc_p(tab_t, w3).reshape(PROWS, D)

    head = pl.kernel(
        _head_body,
        out_type=jax.ShapeDtypeStruct((B,), jnp.float32),
        mesh=mesh,
        scratch_types=[
            pltpu.VMEM((XROW, EPW), jnp.int32),
            pltpu.VMEM((16,), jnp.float32),
            pltpu.VMEM((EPW,), jnp.float32),
            pltpu.VMEM((EPW,), jnp.float32),
            pltpu.VMEM((F * CH,), jnp.int32),
            pltpu.VMEM((F * CH,), jnp.int32),
            pltpu.VMEM((F * CH, D), jnp.float32),
            pltpu.VMEM((F * CH, D), jnp.float32),
            pltpu.VMEM((EPW,), jnp.float32),
            pltpu.SemaphoreType.DMA,
            pltpu.SemaphoreType.DMA,
        ],
        **_SC_PARAMS,
    )
    return head(x_t, p, sq, n0, b16)


def kernel(X, tables, var_table, W, b):
    tab_t = jnp.transpose(tables, (0, 2, 1))
    var_t = var_table.T
    x_t = X.T
    w2 = W.reshape(F + 1, D)
    b16 = jnp.broadcast_to(b.astype(jnp.float32), (16,))
    out = _run(x_t, tab_t, var_t, w2, b16)
    return out.reshape(B, 1)


# speedup vs baseline: 1.1181x; 1.0078x over previous
import jax
import jax.numpy as jnp
from jax import lax
from jax.experimental import pallas as pl
from jax.experimental.pallas import tpu as pltpu
from jax.experimental.pallas import tpu_sc as plsc

B = 4096
F = 26
V = 100000
D = 16
L = 50

VP = 100352
VBLK = 100352
NVB = VP // VBLK
PROWS = F * VP // D

NC = 2
NS = 16
NW = NC * NS
EPW = B // NW
CH = 16
NCHUNK = EPW // CH
XROW = F + L

_SC_PARAMS = dict(
    compiler_params=pltpu.CompilerParams(
        needs_layout_passes=False, use_tc_tiling_on_sc=False))


def _p_body(t_ref, w_ref, o_ref):
    r0 = jnp.dot(w_ref[0], t_ref[0], preferred_element_type=jnp.float32)
    r1 = jnp.dot(w_ref[1], t_ref[1], preferred_element_type=jnp.float32)
    o_ref[...] = jnp.concatenate(
        [r0.reshape(VP // 128, 128), r1.reshape(VP // 128, 128)], axis=0)


def _tc_p(tab_t, w3):
    return pl.pallas_call(
        _p_body,
        grid=(F // 2,),
        in_specs=[
            pl.BlockSpec((2, D, VP), lambda f: (f, 0, 0)),
            pl.BlockSpec((2, 1, D), lambda f: (f, 0, 0)),
        ],
        out_specs=pl.BlockSpec((2 * VP // 128, 128), lambda f: (f, 0)),
        out_shape=jax.ShapeDtypeStruct((F * VP // 128, 128), jnp.float32),
    )(tab_t, w3)


def _q_body(t_ref, w_ref, o_ref):
    t = t_ref[...]
    w = w_ref[0]
    o_ref[...] = jnp.dot(w, t, preferred_element_type=jnp.float32
                         ).reshape(VBLK // 128, 128)


def _tc_q(var_t, w3):
    return pl.pallas_call(
        _q_body,
        grid=(NVB,),
        in_specs=[
            pl.BlockSpec((D, VBLK), lambda k: (0, k)),
            pl.BlockSpec((1, 1, D), lambda k: (F, 0, 0)),
        ],
        out_specs=pl.BlockSpec((VBLK // 128, 128), lambda k: (k, 0)),
        out_shape=jax.ShapeDtypeStruct((VP // 128, 128), jnp.float32),
    )(var_t, w3)


def _pool_body(x_hbm, q_hbm, sq_hbm, n0_hbm, xbuf, qbuf, sqbuf, n0buf, sem_q):
    wid = lax.axis_index("s") * NC + lax.axis_index("c")
    base = pl.multiple_of(wid * EPW, EPW)

    q_cp = pltpu.make_async_copy(q_hbm, qbuf, sem_q)
    q_cp.start()
    pltpu.sync_copy(x_hbm.at[:, pl.ds(base, EPW)], xbuf)
    q_cp.wait()

    lanes = lax.iota(jnp.int32, 16)

    def chunk(c, _):
        e0 = pl.multiple_of(c * CH, CH)
        elane = e0 + lanes

        def qstep(l, carry):
            sq, n0 = carry
            xv = plsc.load_gather(
                xbuf, [jnp.full((16,), F + l, jnp.int32), elane])
            val = plsc.load_gather(qbuf, [xv])
            live = xv != 0
            sq = sq + jnp.where(live, val, 0.0)
            n0 = n0 + jnp.where(live, 0.0, 1.0)
            return sq, n0
        sq, n0 = lax.fori_loop(
            0, L, qstep,
            (jnp.zeros((16,), jnp.float32), jnp.zeros((16,), jnp.float32)))
        sqbuf[pl.ds(e0, CH)] = sq
        n0buf[pl.ds(e0, CH)] = n0
        return _

    lax.fori_loop(0, NCHUNK, chunk, None)
    pltpu.sync_copy(sqbuf, sq_hbm.at[pl.ds(base, EPW)])
    pltpu.sync_copy(n0buf, n0_hbm.at[pl.ds(base, EPW)])


def _head_body(x_hbm, p_hbm, sq_hbm, n0_hbm, b_hbm, out_hbm,
               xbuf, bbuf, sqbuf, n0buf, sidx0, sidx1, srows0, srows1,
               outbuf, sem0, sem1):
    wid = lax.axis_index("s") * NC + lax.axis_index("c")
    base = pl.multiple_of(wid * EPW, EPW)

    pltpu.sync_copy(x_hbm.at[:, pl.ds(base, EPW)], xbuf)
    pltpu.sync_copy(sq_hbm.at[pl.ds(base, EPW)], sqbuf)
    pltpu.sync_copy(n0_hbm.at[pl.ds(base, EPW)], n0buf)
    pltpu.sync_copy(b_hbm, bbuf)

    lanes = lax.iota(jnp.int32, 16)
    bvec = bbuf[...]
    sidx = (sidx0, sidx1)
    srows = (srows0, srows1)
    sems = (sem0, sem1)

    def build(c):
        elane = c * CH + lanes

        def sfill(f, _):
            xv = plsc.load_gather(xbuf, [jnp.full((16,), f, jnp.int32), elane])
            sidx[c % 2][pl.ds(pl.multiple_of(f * CH, CH), CH)] = \
                f * (VP // D) + lax.shift_right_logical(xv, 4)
            return _
        lax.fori_loop(0, F, sfill, None)
        cp = pltpu.make_async_copy(p_hbm.at[sidx[c % 2]], srows[c % 2],
                                   sems[c % 2])
        cp.start()
        return cp

    cps = [None, None]
    cps[0] = build(0)
    for c in range(NCHUNK):
        if c + 1 < NCHUNK:
            cps[(c + 1) % 2] = build(c + 1)
        e0 = pl.multiple_of(c * CH, CH)
        elane = e0 + lanes

        cps[c % 2].wait()

        def pstep(f, sp):
            xv = plsc.load_gather(xbuf, [jnp.full((16,), f, jnp.int32), elane])
            val = plsc.load_gather(
                srows[c % 2], [f * CH + lanes, jnp.bitwise_and(xv, D - 1)])
            return sp + val
        sp = lax.fori_loop(0, F, pstep, jnp.zeros((16,), jnp.float32))

        cnt = jnp.float32(L) - n0buf[pl.ds(e0, CH)]
        logit = sp + sqbuf[pl.ds(e0, CH)] / (cnt + 1e-8) + bvec
        outbuf[pl.ds(e0, CH)] = 1.0 / (1.0 + jnp.exp(-logit))

    pltpu.sync_copy(outbuf, out_hbm.at[pl.ds(base, EPW)])


@jax.jit
def _run(x_t, tab_t, var_t, w2, b16):
    w3 = w2.reshape(F + 1, 1, D)
    w3p = jnp.concatenate(
        [w3, jnp.zeros((1, 1, D), jnp.float32)], axis=0)
    q = _tc_q(var_t, w3).reshape(VP)

    mesh = plsc.VectorSubcoreMesh(core_axis_name="c", subcore_axis_name="s")
    pool = pl.kernel(
        _pool_body,
        out_type=(jax.ShapeDtypeStruct((B,), jnp.float32),
                  jax.ShapeDtypeStruct((B,), jnp.float32)),
        mesh=mesh,
        scratch_types=[
            pltpu.VMEM((XROW, EPW), jnp.int32),
            pltpu.VMEM((VP,), jnp.float32),
            pltpu.VMEM((EPW,), jnp.float32),
            pltpu.VMEM((EPW,), jnp.float32),
            pltpu.SemaphoreType.DMA,
        ],
        **_SC_PARAMS,
    )
    sq, n0 = pool(x_t, q)

    p = _tc_p(tab_t, w3p).reshape(PROWS, D)

    head = pl.kernel(
        _head_body,
        out_type=jax.ShapeDtypeStruct((B,), jnp.float32),
        mesh=mesh,
        scratch_types=[
            pltpu.VMEM((XROW, EPW), jnp.int32),
            pltpu.VMEM((16,), jnp.float32),
            pltpu.VMEM((EPW,), jnp.float32),
            pltpu.VMEM((EPW,), jnp.float32),
            pltpu.VMEM((F * CH,), jnp.int32),
            pltpu.VMEM((F * CH,), jnp.int32),
            pltpu.VMEM((F * CH, D), jnp.float32),
            pltpu.VMEM((F * CH, D), jnp.float32),
            pltpu.VMEM((EPW,), jnp.float32),
            pltpu.SemaphoreType.DMA,
            pltpu.SemaphoreType.DMA,
        ],
        **_SC_PARAMS,
    )
    return head(x_t, p, sq, n0, b16)


def kernel(X, tables, var_table, W, b):
    tab_t = jnp.transpose(tables, (0, 2, 1))
    var_t = var_table.T
    x_t = X.T
    w2 = W.reshape(F + 1, D)
    b16 = jnp.broadcast_to(b.astype(jnp.float32), (16,))
    out = _run(x_t, tab_t, var_t, w2, b16)
    return out.reshape(B, 1)
